# Initial kernel scaffold; baseline (speedup 1.0000x reference)
import jax
import jax.numpy as jnp
from jax.experimental import pallas as pl


def kernel(feats, edge_index, W, b, prelu_a):
    raise NotImplementedError("write your pallas kernel here")


# speedup vs baseline: 5.7343x; 5.7343x over previous
import functools

import jax
import jax.numpy as jnp
from jax import lax
from jax.experimental import pallas as pl
from jax.experimental.pallas import tpu as pltpu
from jax.experimental.pallas import tpu_sc as plsc

N_NODES = 10000
N_EDGES = 320000
D = 128

NC = 2
NS = 16
NW = NC * NS

NPAD = 10240
ROWS_PER_TILE = NPAD // NS
CHUNK = 128
K = -(-N_EDGES // (NW * CHUNK))
EPAD = NW * K * CHUNK

_MESH = plsc.VectorSubcoreMesh(core_axis_name="c", subcore_axis_name="s",
                               num_cores=NC, num_subcores=NS)


def _deg_body(src_hbm, dst_hbm, dego_hbm, degi_hbm, sidx, didx, ho, hi, sem):
    cid = lax.axis_index("c")
    sid = lax.axis_index("s")
    wid = sid * NC + cid

    pltpu.async_copy(src_hbm.at[wid], sidx, sem).wait()
    pltpu.async_copy(dst_hbm.at[wid], didx, sem).wait()

    zeros16 = jnp.zeros((16,), jnp.float32)

    def zero_body(i, _):
        ho[pl.ds(i * 16, 16)] = zeros16
        hi[pl.ds(i * 16, 16)] = zeros16
        return _

    lax.fori_loop(0, NPAD // 16, zero_body, None)

    ones16 = jnp.ones((16,), jnp.float32)

    def edge_body(j, _):
        for k in range(CHUNK // 16):
            s = sidx[j, pl.ds(k * 16, 16)]
            d = didx[j, pl.ds(k * 16, 16)]
            plsc.addupdate_scatter(ho, [s], ones16)
            plsc.addupdate_scatter(hi, [d], ones16)
        return _

    lax.fori_loop(0, K, edge_body, None)

    pltpu.sync_copy(ho, dego_hbm.at[wid])
    pltpu.sync_copy(hi, degi_hbm.at[wid])


_deg_kernel = pl.kernel(
    _deg_body,
    out_type=(jax.ShapeDtypeStruct((NW, NPAD), jnp.float32),
              jax.ShapeDtypeStruct((NW, NPAD), jnp.float32)),
    mesh=_MESH,
    scratch_types=[
        pltpu.VMEM((K, CHUNK), jnp.int32),
        pltpu.VMEM((K, CHUNK), jnp.int32),
        pltpu.VMEM((NPAD,), jnp.float32),
        pltpu.VMEM((NPAD,), jnp.float32),
        pltpu.SemaphoreType.DMA,
    ],
    compiler_params=pltpu.CompilerParams(needs_layout_passes=False),
)


def _scale_body(feats_ref, degp_ref, h_ref):
    deg = jnp.sum(degp_ref[...], axis=0)
    norm = jnp.where(deg > 0, lax.rsqrt(deg), 0.0)
    h_ref[...] = feats_ref[...] * norm[:, None]


_BLK = 1024


def _scale(feats_p, dego_p):
    return pl.pallas_call(
        _scale_body,
        grid=(NPAD // _BLK,),
        in_specs=[
            pl.BlockSpec((_BLK, D), lambda i: (i, 0)),
            pl.BlockSpec((NW, _BLK), lambda i: (0, i)),
        ],
        out_specs=pl.BlockSpec((_BLK, D), lambda i: (i, 0)),
        out_shape=jax.ShapeDtypeStruct((NPAD, D), jnp.float32),
    )(feats_p, dego_p)


def _agg_body(h_hbm, src_hbm, dst_hbm, zeros_hbm, out_hbm,
              sidx, didx, rows, acc, sem):
    cid = lax.axis_index("c")
    sid = lax.axis_index("s")
    wid = sid * NC + cid

    pltpu.sync_copy(zeros_hbm, acc.at[pl.ds(sid * ROWS_PER_TILE, ROWS_PER_TILE)])
    pltpu.async_copy(src_hbm.at[wid], sidx, sem).wait()
    pltpu.async_copy(dst_hbm.at[wid], didx, sem).wait()
    plsc.subcore_barrier()

    def chunk_body(j, _):
        pltpu.async_copy(h_hbm.at[sidx.at[j]], rows, sem).wait()
        pltpu.sync_copy(rows, acc.at[didx.at[j]], add=True)
        return _

    lax.fori_loop(0, K, chunk_body, None)

    plsc.subcore_barrier()
    pltpu.sync_copy(acc.at[pl.ds(sid * ROWS_PER_TILE, ROWS_PER_TILE)],
                    out_hbm.at[cid, pl.ds(sid * ROWS_PER_TILE, ROWS_PER_TILE)])


_agg_kernel = pl.kernel(
    _agg_body,
    out_type=jax.ShapeDtypeStruct((NC, NPAD, D), jnp.float32),
    mesh=_MESH,
    scratch_types=[
        pltpu.VMEM((K, CHUNK), jnp.int32),
        pltpu.VMEM((K, CHUNK), jnp.int32),
        pltpu.VMEM((CHUNK, D), jnp.float32),
        pltpu.VMEM_SHARED((NPAD, D), jnp.float32),
        pltpu.SemaphoreType.DMA,
    ],
)


def _out_body(a0_ref, a1_ref, degp_ref, w_ref, b_ref, pa_ref, o_ref):
    deg = jnp.sum(degp_ref[...], axis=0)
    norm = jnp.where(deg > 0, lax.rsqrt(deg), 0.0)
    rst = (a0_ref[...] + a1_ref[...]) * norm[:, None]
    o = jnp.dot(rst, w_ref[...], preferred_element_type=jnp.float32) + b_ref[...]
    a = pa_ref[0, 0]
    o_ref[...] = jnp.where(o >= 0, o, a * o)


def _finish(a0, a1, degi_p, W, b2, pa2):
    return pl.pallas_call(
        _out_body,
        grid=(NPAD // _BLK,),
        in_specs=[
            pl.BlockSpec((_BLK, D), lambda i: (i, 0)),
            pl.BlockSpec((_BLK, D), lambda i: (i, 0)),
            pl.BlockSpec((NW, _BLK), lambda i: (0, i)),
            pl.BlockSpec((D, D), lambda i: (0, 0)),
            pl.BlockSpec((1, D), lambda i: (0, 0)),
            pl.BlockSpec((1, 1), lambda i: (0, 0), memory_space=pltpu.SMEM),
        ],
        out_specs=pl.BlockSpec((_BLK, D), lambda i: (i, 0)),
        out_shape=jax.ShapeDtypeStruct((NPAD, D), jnp.float32),
    )(a0, a1, degi_p, W, b2, pa2)


def kernel(feats, edge_index, W, b, prelu_a):
    src = edge_index[0].astype(jnp.int32)
    dst = edge_index[1].astype(jnp.int32)
    pad = jnp.full((EPAD - N_EDGES,), N_NODES, jnp.int32)
    src_p = jnp.concatenate([src, pad]).reshape(NW, K, CHUNK)
    dst_p = jnp.concatenate([dst, pad]).reshape(NW, K, CHUNK)

    feats_p = jnp.pad(feats, ((0, NPAD - N_NODES), (0, 0)))
    zeros_tile = jnp.zeros((ROWS_PER_TILE, D), jnp.float32)

    dego_p, degi_p = _deg_kernel(src_p, dst_p)
    h = _scale(feats_p, dego_p)
    agg = _agg_kernel(h, src_p, dst_p, zeros_tile)
    out = _finish(agg[0], agg[1], degi_p, W,
                  b.reshape(1, D), prelu_a.reshape(1, 1))
    return out[:N_NODES]


# speedup vs baseline: 6.7242x; 1.1726x over previous
import jax
import jax.numpy as jnp
from jax import lax
from jax.experimental import pallas as pl
from jax.experimental.pallas import tpu as pltpu
from jax.experimental.pallas import tpu_sc as plsc

N_NODES = 10000
N_EDGES = 320000
D = 128

NC = 2
NS = 16
NW = NC * NS

CHUNK = 128
K = -(-N_EDGES // (NW * CHUNK))
E_PER_W = K * CHUNK
EPAD = NW * E_PER_W
NPAD = 10240
ROWS_PER_TILE = NPAD // NS

_MESH = plsc.VectorSubcoreMesh(core_axis_name="c", subcore_axis_name="s",
                               num_cores=NC, num_subcores=NS)
_SC_PARAMS = pltpu.CompilerParams(needs_layout_passes=False)


def _deg_body(src_hbm, dst_hbm, dego_hbm, degi_hbm, sidx, didx, ho, hi, sem):
    cid = lax.axis_index("c")
    sid = lax.axis_index("s")
    wid = sid * NC + cid

    pltpu.async_copy(src_hbm.at[wid], sidx, sem).wait()
    pltpu.async_copy(dst_hbm.at[wid], didx, sem).wait()

    zeros16 = jnp.zeros((16,), jnp.float32)

    def zero_body(i, _):
        ho[pl.ds(i * 16, 16)] = zeros16
        hi[pl.ds(i * 16, 16)] = zeros16
        return _

    lax.fori_loop(0, NPAD // 16, zero_body, None)

    ones16 = jnp.ones((16,), jnp.float32)

    def edge_body(i, _):
        s = sidx[pl.ds(i * 16, 16)]
        d = didx[pl.ds(i * 16, 16)]
        plsc.addupdate_scatter(ho, [s], ones16)
        plsc.addupdate_scatter(hi, [d], ones16)
        return _

    lax.fori_loop(0, E_PER_W // 16, edge_body, None)

    pltpu.sync_copy(ho, dego_hbm.at[wid])
    pltpu.sync_copy(hi, degi_hbm.at[wid])


_deg_kernel = pl.kernel(
    _deg_body,
    out_type=(jax.ShapeDtypeStruct((NW, NPAD), jnp.float32),
              jax.ShapeDtypeStruct((NW, NPAD), jnp.float32)),
    mesh=_MESH,
    scratch_types=[
        pltpu.VMEM((E_PER_W,), jnp.int32),
        pltpu.VMEM((E_PER_W,), jnp.int32),
        pltpu.VMEM((NPAD,), jnp.float32),
        pltpu.VMEM((NPAD,), jnp.float32),
        pltpu.SemaphoreType.DMA,
    ],
    compiler_params=_SC_PARAMS,
)


def _scale_body(feats_ref, degp_ref, h_ref):
    deg = jnp.sum(degp_ref[...], axis=0)
    norm = jnp.where(deg > 0, lax.rsqrt(deg), 0.0)
    h_ref[...] = feats_ref[...] * norm[:, None]


def _scale(feats_p, dego_p):
    return pl.pallas_call(
        _scale_body,
        out_shape=jax.ShapeDtypeStruct((NPAD, D), jnp.float32),
    )(feats_p, dego_p)


def _agg_body(h_hbm, src_hbm, dst_hbm, out_hbm,
              sidx, didx, rows, acc, semA):
    cid = lax.axis_index("c")
    sid = lax.axis_index("s")
    wid = sid * NC + cid

    zeros16 = jnp.zeros((16,), jnp.float32)

    def zero_body(r, _):
        for c in range(D // 16):
            rows[r, pl.ds(c * 16, 16)] = zeros16
        return _

    lax.fori_loop(0, CHUNK, zero_body, None)
    for k in range(ROWS_PER_TILE // CHUNK):
        pltpu.sync_copy(rows, acc.at[pl.ds(sid * ROWS_PER_TILE + k * CHUNK, CHUNK)])

    pltpu.async_copy(src_hbm.at[wid], sidx, semA).wait()
    pltpu.async_copy(dst_hbm.at[wid], didx, semA).wait()
    plsc.subcore_barrier()

    def chunk_body(j, _):
        pltpu.async_copy(h_hbm.at[sidx.at[j]], rows, semA).wait()
        pltpu.sync_copy(rows, acc.at[didx.at[j]], add=True)
        return _

    lax.fori_loop(0, K, chunk_body, None)

    plsc.subcore_barrier()
    pltpu.sync_copy(acc.at[pl.ds(sid * ROWS_PER_TILE, ROWS_PER_TILE)],
                    out_hbm.at[cid, pl.ds(sid * ROWS_PER_TILE, ROWS_PER_TILE)])


_agg_kernel = pl.kernel(
    _agg_body,
    out_type=jax.ShapeDtypeStruct((NC, NPAD, D), jnp.float32),
    mesh=_MESH,
    scratch_types=[
        pltpu.VMEM((K, CHUNK), jnp.int32),
        pltpu.VMEM((K, CHUNK), jnp.int32),
        pltpu.VMEM((CHUNK, D), jnp.float32),
        pltpu.VMEM_SHARED((NPAD, D), jnp.float32),
        pltpu.SemaphoreType.DMA,
    ],
    compiler_params=_SC_PARAMS,
)


def _out_body(a0_ref, a1_ref, degp_ref, w_ref, b_ref, pa_ref, o_ref):
    deg = jnp.sum(degp_ref[...], axis=0)
    norm = jnp.where(deg > 0, lax.rsqrt(deg), 0.0)
    rst = (a0_ref[0:N_NODES] + a1_ref[0:N_NODES]) * norm[0:N_NODES, None]
    o = jnp.dot(rst, w_ref[...], preferred_element_type=jnp.float32) + b_ref[...]
    a = pa_ref[0, 0]
    o_ref[...] = jnp.where(o >= 0, o, a * o)


def _finish(a0, a1, degi_p, W, b2, pa2):
    return pl.pallas_call(
        _out_body,
        in_specs=[
            pl.BlockSpec(memory_space=pltpu.VMEM),
            pl.BlockSpec(memory_space=pltpu.VMEM),
            pl.BlockSpec(memory_space=pltpu.VMEM),
            pl.BlockSpec(memory_space=pltpu.VMEM),
            pl.BlockSpec(memory_space=pltpu.VMEM),
            pl.BlockSpec(memory_space=pltpu.SMEM),
        ],
        out_shape=jax.ShapeDtypeStruct((N_NODES, D), jnp.float32),
    )(a0, a1, degi_p, W, b2, pa2)


def kernel(feats, edge_index, W, b, prelu_a):
    src = edge_index[0].astype(jnp.int32)
    dst = edge_index[1].astype(jnp.int32)
    pad = jnp.full((EPAD - N_EDGES,), N_NODES, jnp.int32)
    src_p = jnp.concatenate([src, pad]).reshape(NW, E_PER_W)
    dst_p = jnp.concatenate([dst, pad]).reshape(NW, E_PER_W)
    src_c = src_p.reshape(NW, K, CHUNK)
    dst_c = dst_p.reshape(NW, K, CHUNK)

    feats_p = jnp.pad(feats, ((0, NPAD - N_NODES), (0, 0)))

    dego_p, degi_p = _deg_kernel(src_p, dst_p)
    h = _scale(feats_p, dego_p)
    agg = _agg_kernel(h, src_c, dst_c)
    out = _finish(agg[0], agg[1], degi_p, W,
                  b.reshape(1, D), prelu_a.reshape(1, 1))
    return out
